# baseline (speedup 1.0000x reference)
import functools

import jax
import jax.numpy as jnp
from jax import lax
from jax.experimental import pallas as pl
from jax.experimental.pallas import tpu as pltpu
from jax.experimental.pallas import tpu_sc as plsc

_LANES = 16


@functools.lru_cache(maxsize=None)
def _build(B, C, K):
    NC, NS = 2, 16
    NW = NC * NS
    RW = B // NW
    NG = RW // _LANES
    OUTW = _LANES

    mesh = plsc.VectorSubcoreMesh(core_axis_name="c", subcore_axis_name="s")

    GPI = 4
    NP = NG // GPI

    @functools.partial(
        pl.kernel,
        mesh=mesh,
        out_type=jax.ShapeDtypeStruct((B, OUTW), jnp.float32),
        compiler_params=pltpu.CompilerParams(needs_layout_passes=False),
        scratch_types=[
            pltpu.VMEM((GPI * _LANES, C), jnp.float32),
            pltpu.VMEM((RW,), jnp.int32),
            pltpu.VMEM((RW, OUTW), jnp.float32),
        ],
    )
    def sc_mine(logits_hbm, targets_hbm, out_hbm, buf, tgt_v, out_v):
        wid = lax.axis_index("s") * NC + lax.axis_index("c")
        row0 = wid * RW
        pltpu.sync_copy(targets_hbm.at[pl.ds(row0, RW)], tgt_v)

        iota = lax.iota(jnp.int32, _LANES)
        ninf = jnp.full((_LANES,), -jnp.inf, jnp.float32)
        one = jnp.full((_LANES,), 1, jnp.int32)
        cmax = jnp.full((_LANES,), C, jnp.int32)
        phase = (iota * 9) & 15

        for p in range(NP):
            pltpu.sync_copy(
                logits_hbm.at[pl.ds(row0 + p * GPI * _LANES, GPI * _LANES), :],
                buf,
            )
            tgts = [
                tgt_v[pl.ds((p * GPI + i) * _LANES, _LANES)]
                for i in range(GPI)
            ]

            def body(_, carry, tgts=tgts):
                colv, poss, topss = carry
                new_poss, new_topss = [], []
                for i in range(GPI):
                    v = plsc.load_gather(buf, [iota + i * _LANES, colv])
                    eq = tgts[i] == colv
                    new_poss.append(jnp.where(eq, v, poss[i]))
                    new = jnp.where(eq, ninf, v)
                    tops = topss[i]
                    upd = [jnp.maximum(tops[0], new)]
                    for j in range(1, K):
                        upd.append(
                            jnp.maximum(tops[j], jnp.minimum(tops[j - 1], new))
                        )
                    new_topss.append(tuple(upd))
                colv = colv + one
                colv = jnp.where(colv == cmax, 0, colv)
                return colv, tuple(new_poss), tuple(new_topss)

            _, poss, topss = lax.fori_loop(
                0, C, body,
                (phase, (ninf,) * GPI, ((ninf,) * K,) * GPI),
            )

            for i in range(GPI):
                rows = (p * GPI + i) * _LANES + iota
                plsc.store_scatter(out_v, [rows, jnp.full((_LANES,), 0, jnp.int32)], poss[i])
                for j in range(1, K):
                    plsc.store_scatter(out_v, [rows, jnp.full((_LANES,), j, jnp.int32)], topss[i][j])
                for j in range(K, OUTW):
                    plsc.store_scatter(out_v, [rows, jnp.full((_LANES,), j, jnp.int32)], ninf)

        pltpu.sync_copy(out_v, out_hbm.at[pl.ds(row0, RW), :])

    def tc_finish(res_ref, out_ref):
        x = res_ref[...] * 10.0
        m = jnp.max(x, axis=1, keepdims=True)
        s = jnp.sum(jnp.exp(x - m), axis=1)
        lse = m[:, 0] + jnp.log(s)
        loss = lse - x[:, 0]
        out_ref[...] = (jnp.sum(loss) * (1.0 / B)).reshape(1, 1)

    tc_call = pl.pallas_call(
        tc_finish,
        out_shape=jax.ShapeDtypeStruct((1, 1), jnp.float32),
    )

    def run(logits, targets):
        res = sc_mine(logits, targets)
        return tc_call(res)[0, 0]

    return run


def kernel(logits, targets):
    B, C = logits.shape
    K = int(0.01 * (C - 1))
    return _build(B, C, K)(logits, targets.astype(jnp.int32))


# speedup vs baseline: 1.2295x; 1.2295x over previous
import functools

import jax
import jax.numpy as jnp
from jax import lax
from jax.experimental import pallas as pl
from jax.experimental.pallas import tpu as pltpu
from jax.experimental.pallas import tpu_sc as plsc

_LANES = 16


@functools.lru_cache(maxsize=None)
def _build(B, C, K):
    NC, NS = 2, 16
    NW = NC * NS
    RW = B // NW
    NG = RW // _LANES
    OUTW = _LANES

    mesh = plsc.VectorSubcoreMesh(core_axis_name="c", subcore_axis_name="s")

    GPI = 2
    NP = NG // GPI

    @functools.partial(
        pl.kernel,
        mesh=mesh,
        out_type=jax.ShapeDtypeStruct((B, OUTW), jnp.float32),
        compiler_params=pltpu.CompilerParams(needs_layout_passes=False),
        scratch_types=[
            pltpu.VMEM((GPI * _LANES, C), jnp.float32),
            pltpu.VMEM((RW,), jnp.int32),
            pltpu.VMEM((RW, OUTW), jnp.float32),
        ],
    )
    def sc_mine(logits_hbm, targets_hbm, out_hbm, buf, tgt_v, out_v):
        wid = lax.axis_index("s") * NC + lax.axis_index("c")
        row0 = wid * RW
        pltpu.sync_copy(targets_hbm.at[pl.ds(row0, RW)], tgt_v)

        iota = lax.iota(jnp.int32, _LANES)
        ninf = jnp.full((_LANES,), -jnp.inf, jnp.float32)
        one = jnp.full((_LANES,), 1, jnp.int32)
        cmax = jnp.full((_LANES,), C, jnp.int32)
        phase = (iota * 9) & 15

        for p in range(NP):
            pltpu.sync_copy(
                logits_hbm.at[pl.ds(row0 + p * GPI * _LANES, GPI * _LANES), :],
                buf,
            )
            tgts = [
                tgt_v[pl.ds((p * GPI + i) * _LANES, _LANES)]
                for i in range(GPI)
            ]

            def body(_, carry, tgts=tgts):
                colv, poss, topss = carry
                new_poss, new_topss = [], []
                for i in range(GPI):
                    v = plsc.load_gather(buf, [iota + i * _LANES, colv])
                    eq = tgts[i] == colv
                    new_poss.append(jnp.where(eq, v, poss[i]))
                    new = jnp.where(eq, ninf, v)
                    tops = topss[i]
                    upd = [jnp.maximum(tops[0], new)]
                    for j in range(1, K):
                        upd.append(
                            jnp.maximum(tops[j], jnp.minimum(tops[j - 1], new))
                        )
                    new_topss.append(tuple(upd))
                colv = colv + one
                colv = jnp.where(colv == cmax, 0, colv)
                return colv, tuple(new_poss), tuple(new_topss)

            _, poss, topss = lax.fori_loop(
                0, C, body,
                (phase, (ninf,) * GPI, ((ninf,) * K,) * GPI),
            )

            for i in range(GPI):
                rows = (p * GPI + i) * _LANES + iota
                plsc.store_scatter(out_v, [rows, jnp.full((_LANES,), 0, jnp.int32)], poss[i])
                for j in range(1, K):
                    plsc.store_scatter(out_v, [rows, jnp.full((_LANES,), j, jnp.int32)], topss[i][j])
                for j in range(K, OUTW):
                    plsc.store_scatter(out_v, [rows, jnp.full((_LANES,), j, jnp.int32)], ninf)

        pltpu.sync_copy(out_v, out_hbm.at[pl.ds(row0, RW), :])

    def tc_finish(res_ref, out_ref):
        x = res_ref[...] * 10.0
        m = jnp.max(x, axis=1, keepdims=True)
        s = jnp.sum(jnp.exp(x - m), axis=1)
        lse = m[:, 0] + jnp.log(s)
        loss = lse - x[:, 0]
        out_ref[...] = (jnp.sum(loss) * (1.0 / B)).reshape(1, 1)

    tc_call = pl.pallas_call(
        tc_finish,
        out_shape=jax.ShapeDtypeStruct((1, 1), jnp.float32),
    )

    def run(logits, targets):
        res = sc_mine(logits, targets)
        return tc_call(res)[0, 0]

    return run


def kernel(logits, targets):
    B, C = logits.shape
    K = int(0.01 * (C - 1))
    return _build(B, C, K)(logits, targets.astype(jnp.int32))


# speedup vs baseline: 1.6251x; 1.3218x over previous
import functools

import jax
import jax.numpy as jnp
from jax import lax
from jax.experimental import pallas as pl
from jax.experimental.pallas import tpu as pltpu
from jax.experimental.pallas import tpu_sc as plsc

_LANES = 16


@functools.lru_cache(maxsize=None)
def _build(B, C, K):
    NC, NS = 2, 16
    NW = NC * NS
    RW = B // NW
    OUTW = _LANES

    RI = 4
    PASS_ROWS = 64
    NPASS = RW // PASS_ROWS
    NQ = PASS_ROWS // RI
    NF = C // _LANES
    REM = C - NF * _LANES

    mesh = plsc.VectorSubcoreMesh(core_axis_name="c", subcore_axis_name="s")

    @functools.partial(
        pl.kernel,
        mesh=mesh,
        out_type=jax.ShapeDtypeStruct((B, OUTW), jnp.float32),
        compiler_params=pltpu.CompilerParams(needs_layout_passes=False),
        scratch_types=[
            pltpu.VMEM((PASS_ROWS, C), jnp.float32),
            pltpu.VMEM((RW,), jnp.int32),
            pltpu.VMEM((RW, OUTW), jnp.float32),
        ],
    )
    def sc_mine(logits_hbm, targets_hbm, out_hbm, buf, tgt_v, out_v):
        wid = lax.axis_index("s") * NC + lax.axis_index("c")
        row0 = wid * RW
        pltpu.sync_copy(targets_hbm.at[pl.ds(row0, RW)], tgt_v)

        iota = lax.iota(jnp.int32, _LANES)
        ninf = jnp.full((_LANES,), -jnp.inf, jnp.float32)

        def merge(t, v):
            vd, _ = plsc.sort_key_val(v, v, descending=True)
            m = jnp.maximum(t, vd)
            ts, _ = plsc.sort_key_val(m, m)
            return ts

        for p in range(NPASS):
            pltpu.sync_copy(
                logits_hbm.at[pl.ds(row0 + p * PASS_ROWS, PASS_ROWS), :],
                buf,
            )

            def quad_body(q, _, p=p):
                rb = q * RI
                tspl = [
                    plsc.load_gather(
                        tgt_v,
                        [jnp.full((_LANES,), p * PASS_ROWS + rb + i, jnp.int32)],
                    )
                    for i in range(RI)
                ]

                def jbody(j, ts):
                    cols = jnp.full((_LANES,), j * _LANES, jnp.int32) + iota
                    out = []
                    for i in range(RI):
                        v = buf[rb + i, pl.ds(j * _LANES, _LANES)]
                        v = jnp.where(cols == tspl[i], ninf, v)
                        out.append(merge(ts[i], v))
                    return tuple(out)

                ts = lax.fori_loop(0, NF, jbody, (ninf,) * RI)

                if REM:
                    colst = jnp.full((_LANES,), C - _LANES, jnp.int32) + iota
                    tail = []
                    for i in range(RI):
                        v = buf[rb + i, pl.ds(C - _LANES, _LANES)]
                        v = jnp.where(iota < _LANES - REM, ninf, v)
                        v = jnp.where(colst == tspl[i], ninf, v)
                        tail.append(merge(ts[i], v))
                    ts = tuple(tail)

                for i in range(RI):
                    rev = lax.rev(ts[i], (0,))
                    posv = plsc.load_gather(
                        buf, [jnp.full((_LANES,), rb + i, jnp.int32), tspl[i]]
                    )
                    row_vec = jnp.where(
                        iota == 0, posv, jnp.where(iota < K, rev, ninf)
                    )
                    out_v[p * PASS_ROWS + rb + i, :] = row_vec
                return 0

            lax.fori_loop(0, NQ, quad_body, 0)

        pltpu.sync_copy(out_v, out_hbm.at[pl.ds(row0, RW), :])

    def tc_finish(res_ref, out_ref):
        x = res_ref[...] * 10.0
        m = jnp.max(x, axis=1, keepdims=True)
        s = jnp.sum(jnp.exp(x - m), axis=1)
        lse = m[:, 0] + jnp.log(s)
        loss = lse - x[:, 0]
        out_ref[...] = (jnp.sum(loss) * (1.0 / B)).reshape(1, 1)

    tc_call = pl.pallas_call(
        tc_finish,
        out_shape=jax.ShapeDtypeStruct((1, 1), jnp.float32),
    )

    def run(logits, targets):
        res = sc_mine(logits, targets)
        return tc_call(res)[0, 0]

    return run


def kernel(logits, targets):
    B, C = logits.shape
    K = int(0.01 * (C - 1))
    return _build(B, C, K)(logits, targets.astype(jnp.int32))


# speedup vs baseline: 1.8589x; 1.1439x over previous
import functools

import jax
import jax.numpy as jnp
from jax import lax
from jax.experimental import pallas as pl
from jax.experimental.pallas import tpu as pltpu
from jax.experimental.pallas import tpu_sc as plsc

_LANES = 16


@functools.lru_cache(maxsize=None)
def _build(B, C, K):
    NC, NS = 2, 16
    NW = NC * NS
    RW = B // NW
    OUTW = _LANES

    RI = 8
    PASS_ROWS = 64
    NPASS = RW // PASS_ROWS
    NQ = PASS_ROWS // RI
    NF = C // _LANES
    REM = C - NF * _LANES

    mesh = plsc.VectorSubcoreMesh(core_axis_name="c", subcore_axis_name="s")

    @functools.partial(
        pl.kernel,
        mesh=mesh,
        out_type=jax.ShapeDtypeStruct((B, OUTW), jnp.float32),
        compiler_params=pltpu.CompilerParams(needs_layout_passes=False),
        scratch_types=[
            pltpu.VMEM((PASS_ROWS, C), jnp.float32),
            pltpu.VMEM((RW,), jnp.int32),
            pltpu.VMEM((RW, OUTW), jnp.float32),
        ],
    )
    def sc_mine(logits_hbm, targets_hbm, out_hbm, buf, tgt_v, out_v):
        wid = lax.axis_index("s") * NC + lax.axis_index("c")
        row0 = wid * RW
        pltpu.sync_copy(targets_hbm.at[pl.ds(row0, RW)], tgt_v)

        iota = lax.iota(jnp.int32, _LANES)
        ninf = jnp.full((_LANES,), -jnp.inf, jnp.float32)

        def merge(t, v):
            vd, _ = plsc.sort_key_val(v, v, descending=True)
            m = jnp.maximum(t, vd)
            ts, _ = plsc.sort_key_val(m, m)
            return ts

        for p in range(NPASS):
            pltpu.sync_copy(
                logits_hbm.at[pl.ds(row0 + p * PASS_ROWS, PASS_ROWS), :],
                buf,
            )

            def quad_body(q, _, p=p):
                rb = q * RI
                tspl = [
                    plsc.load_gather(
                        tgt_v,
                        [jnp.full((_LANES,), p * PASS_ROWS + rb + i, jnp.int32)],
                    )
                    for i in range(RI)
                ]

                def jbody(j, ts):
                    cols = jnp.full((_LANES,), j * _LANES, jnp.int32) + iota
                    out = []
                    for i in range(RI):
                        v = buf[rb + i, pl.ds(j * _LANES, _LANES)]
                        v = jnp.where(cols == tspl[i], ninf, v)
                        out.append(merge(ts[i], v))
                    return tuple(out)

                ts = lax.fori_loop(0, NF, jbody, (ninf,) * RI)

                if REM:
                    colst = jnp.full((_LANES,), C - _LANES, jnp.int32) + iota
                    tail = []
                    for i in range(RI):
                        v = buf[rb + i, pl.ds(C - _LANES, _LANES)]
                        v = jnp.where(iota < _LANES - REM, ninf, v)
                        v = jnp.where(colst == tspl[i], ninf, v)
                        tail.append(merge(ts[i], v))
                    ts = tuple(tail)

                for i in range(RI):
                    rev = lax.rev(ts[i], (0,))
                    posv = plsc.load_gather(
                        buf, [jnp.full((_LANES,), rb + i, jnp.int32), tspl[i]]
                    )
                    row_vec = jnp.where(
                        iota == 0, posv, jnp.where(iota < K, rev, ninf)
                    )
                    out_v[p * PASS_ROWS + rb + i, :] = row_vec
                return 0

            lax.fori_loop(0, NQ, quad_body, 0)

        pltpu.sync_copy(out_v, out_hbm.at[pl.ds(row0, RW), :])

    def tc_finish(res_ref, out_ref):
        x = res_ref[...] * 10.0
        m = jnp.max(x, axis=1, keepdims=True)
        s = jnp.sum(jnp.exp(x - m), axis=1)
        lse = m[:, 0] + jnp.log(s)
        loss = lse - x[:, 0]
        out_ref[...] = (jnp.sum(loss) * (1.0 / B)).reshape(1, 1)

    tc_call = pl.pallas_call(
        tc_finish,
        out_shape=jax.ShapeDtypeStruct((1, 1), jnp.float32),
    )

    def run(logits, targets):
        res = sc_mine(logits, targets)
        return tc_call(res)[0, 0]

    return run


def kernel(logits, targets):
    B, C = logits.shape
    K = int(0.01 * (C - 1))
    return _build(B, C, K)(logits, targets.astype(jnp.int32))


# speedup vs baseline: 1.8686x; 1.0052x over previous
import functools

import jax
import jax.numpy as jnp
from jax import lax
from jax.experimental import pallas as pl
from jax.experimental.pallas import tpu as pltpu
from jax.experimental.pallas import tpu_sc as plsc

_LANES = 16


@functools.lru_cache(maxsize=None)
def _build(B, C, K):
    NC, NS = 2, 16
    NW = NC * NS
    RW = B // NW
    OUTW = _LANES

    RI = 16
    PASS_ROWS = 64
    NPASS = RW // PASS_ROWS
    NQ = PASS_ROWS // RI
    NF = C // _LANES
    REM = C - NF * _LANES

    mesh = plsc.VectorSubcoreMesh(core_axis_name="c", subcore_axis_name="s")

    @functools.partial(
        pl.kernel,
        mesh=mesh,
        out_type=jax.ShapeDtypeStruct((B, OUTW), jnp.float32),
        compiler_params=pltpu.CompilerParams(needs_layout_passes=False),
        scratch_types=[
            pltpu.VMEM((PASS_ROWS, C), jnp.float32),
            pltpu.VMEM((RW,), jnp.int32),
            pltpu.VMEM((RW, OUTW), jnp.float32),
        ],
    )
    def sc_mine(logits_hbm, targets_hbm, out_hbm, buf, tgt_v, out_v):
        wid = lax.axis_index("s") * NC + lax.axis_index("c")
        row0 = wid * RW
        pltpu.sync_copy(targets_hbm.at[pl.ds(row0, RW)], tgt_v)

        iota = lax.iota(jnp.int32, _LANES)
        ninf = jnp.full((_LANES,), -jnp.inf, jnp.float32)

        def merge(t, v):
            vd, _ = plsc.sort_key_val(v, v, descending=True)
            m = jnp.maximum(t, vd)
            ts, _ = plsc.sort_key_val(m, m)
            return ts

        for p in range(NPASS):
            pltpu.sync_copy(
                logits_hbm.at[pl.ds(row0 + p * PASS_ROWS, PASS_ROWS), :],
                buf,
            )

            def quad_body(q, _, p=p):
                rb = q * RI
                tspl = [
                    plsc.load_gather(
                        tgt_v,
                        [jnp.full((_LANES,), p * PASS_ROWS + rb + i, jnp.int32)],
                    )
                    for i in range(RI)
                ]

                def jbody(j, ts):
                    cols = jnp.full((_LANES,), j * _LANES, jnp.int32) + iota
                    out = []
                    for i in range(RI):
                        v = buf[rb + i, pl.ds(j * _LANES, _LANES)]
                        v = jnp.where(cols == tspl[i], ninf, v)
                        out.append(merge(ts[i], v))
                    return tuple(out)

                ts = lax.fori_loop(0, NF, jbody, (ninf,) * RI)

                if REM:
                    colst = jnp.full((_LANES,), C - _LANES, jnp.int32) + iota
                    tail = []
                    for i in range(RI):
                        v = buf[rb + i, pl.ds(C - _LANES, _LANES)]
                        v = jnp.where(iota < _LANES - REM, ninf, v)
                        v = jnp.where(colst == tspl[i], ninf, v)
                        tail.append(merge(ts[i], v))
                    ts = tuple(tail)

                for i in range(RI):
                    rev = lax.rev(ts[i], (0,))
                    posv = plsc.load_gather(
                        buf, [jnp.full((_LANES,), rb + i, jnp.int32), tspl[i]]
                    )
                    row_vec = jnp.where(
                        iota == 0, posv, jnp.where(iota < K, rev, ninf)
                    )
                    out_v[p * PASS_ROWS + rb + i, :] = row_vec
                return 0

            lax.fori_loop(0, NQ, quad_body, 0)

        pltpu.sync_copy(out_v, out_hbm.at[pl.ds(row0, RW), :])

    def tc_finish(res_ref, out_ref):
        x = res_ref[...] * 10.0
        m = jnp.max(x, axis=1, keepdims=True)
        s = jnp.sum(jnp.exp(x - m), axis=1)
        lse = m[:, 0] + jnp.log(s)
        loss = lse - x[:, 0]
        out_ref[...] = (jnp.sum(loss) * (1.0 / B)).reshape(1, 1)

    tc_call = pl.pallas_call(
        tc_finish,
        out_shape=jax.ShapeDtypeStruct((1, 1), jnp.float32),
    )

    def run(logits, targets):
        res = sc_mine(logits, targets)
        return tc_call(res)[0, 0]

    return run


def kernel(logits, targets):
    B, C = logits.shape
    K = int(0.01 * (C - 1))
    return _build(B, C, K)(logits, targets.astype(jnp.int32))
